# baseline (speedup 1.0000x reference)
import functools

import jax
import jax.numpy as jnp
import numpy as np
from jax import lax
from jax.experimental import pallas as pl
from jax.experimental.pallas import tpu as pltpu
from jax.experimental.pallas import tpu_sc as plsc

F = 26
V = 1000
TOT = F * V
D = 16
B = 4096
NP0 = F * (F - 1) // 2
NPP = 328
FP = 32
ZROW = TOT
_PI = np.array([i for i in range(F - 1) for j in range(i + 1, F)], dtype=np.int32)
_PJ = np.array([j for i in range(F - 1) for j in range(i + 1, F)], dtype=np.int32)

NC, NS = 2, 16
NW = NC * NS
PROWS = 2 * B * NPP
PRPW = PROWS // NW
PCHUNK = 2624
PNCHUNK = PRPW // PCHUNK
LROWS = B * FP
LRPW = LROWS // NW
LCHUNK = 2048
LNCHUNK = LRPW // LCHUNK


def _sc_gather(tflat, linpad, idxp, idxl):
    @functools.partial(
        pl.kernel,
        out_type=(
            jax.ShapeDtypeStruct((PROWS, D), jnp.float32),
            jax.ShapeDtypeStruct((LROWS, D), jnp.float32),
        ),
        mesh=plsc.VectorSubcoreMesh(core_axis_name="c", subcore_axis_name="s",
                                    num_cores=NC, num_subcores=NS),
        compiler_params=pltpu.CompilerParams(use_tc_tiling_on_sc=False),
        scratch_types=[
            pltpu.VMEM((PCHUNK,), jnp.int32),
            pltpu.VMEM((PCHUNK, D), jnp.float32),
            pltpu.VMEM((LCHUNK,), jnp.int32),
            pltpu.VMEM((LCHUNK, D), jnp.float32),
            pltpu.SemaphoreType.DMA,
        ],
    )
    def body(tflat_hbm, linpad_hbm, idxp_hbm, idxl_hbm, pall_hbm, plin_hbm,
             idx_v, rows_v, idxl_v, rowsl_v, sem):
        wid = lax.axis_index("s") * NC + lax.axis_index("c")
        base0 = wid * PRPW
        for c in range(PNCHUNK):
            base = base0 + c * PCHUNK
            pltpu.sync_copy(idxp_hbm.at[pl.ds(base, PCHUNK)], idx_v)
            pltpu.async_copy(tflat_hbm.at[idx_v], rows_v, sem).wait()
            pltpu.sync_copy(rows_v, pall_hbm.at[pl.ds(base, PCHUNK)])
        lbase0 = wid * LRPW
        for c in range(LNCHUNK):
            base = lbase0 + c * LCHUNK
            pltpu.sync_copy(idxl_hbm.at[pl.ds(base, LCHUNK)], idxl_v)
            pltpu.async_copy(linpad_hbm.at[idxl_v], rowsl_v, sem).wait()
            pltpu.sync_copy(rowsl_v, plin_hbm.at[pl.ds(base, LCHUNK)])

    return body(tflat, linpad, idxp, idxl)


def _tc_mlp(p1, p2, plin, w1t, b1r, w2t, b2r, w3t, b3r):
    BB = 32
    NB = B // BB

    def body(p1_ref, p2_ref, plin_ref, w1_ref, b1_ref, w2_ref, b2_ref,
             w3_ref, b3_ref, s1_ref, s2_ref):
        ix = p1_ref[...] * p2_ref[...]
        ix3 = ix.reshape(BB, NPP, D)
        pp = lax.broadcasted_iota(jnp.int32, (BB, NPP, D), 1)
        ixm = jnp.where(pp >= NP0, 0.0, ix3)
        ffm = jnp.sum(jnp.sum(ixm, axis=1), axis=1, keepdims=True)
        lin3 = plin_ref[...].reshape(BB, FP, D)
        lin = jnp.sum(jnp.sum(lin3, axis=1), axis=1, keepdims=True)
        xmat = ixm.reshape(BB * NPP, D)
        h1 = jax.nn.relu(
            jnp.dot(xmat, w1_ref[...], preferred_element_type=jnp.float32)
            + b1_ref[...])
        h2 = jax.nn.relu(
            jnp.dot(h1, w2_ref[...], preferred_element_type=jnp.float32)
            + b2_ref[...])
        fi = (jnp.dot(h2, w3_ref[...], preferred_element_type=jnp.float32)
              + b3_ref[...])
        fi3 = fi.reshape(BB, NPP, 1)
        fisum = jnp.sum(fi3, axis=1)
        s1_ref[...] = lin + ffm
        s2_ref[...] = fisum

    return pl.pallas_call(
        body,
        grid=(NB,),
        in_specs=[
            pl.BlockSpec((BB * NPP, D), lambda i: (i, 0)),
            pl.BlockSpec((BB * NPP, D), lambda i: (i, 0)),
            pl.BlockSpec((BB * FP, D), lambda i: (i, 0)),
            pl.BlockSpec((D, 64), lambda i: (0, 0)),
            pl.BlockSpec((1, 64), lambda i: (0, 0)),
            pl.BlockSpec((64, 32), lambda i: (0, 0)),
            pl.BlockSpec((1, 32), lambda i: (0, 0)),
            pl.BlockSpec((32, 1), lambda i: (0, 0)),
            pl.BlockSpec((1, 1), lambda i: (0, 0)),
        ],
        out_specs=[
            pl.BlockSpec((BB, 1), lambda i: (i, 0)),
            pl.BlockSpec((BB, 1), lambda i: (i, 0)),
        ],
        out_shape=[
            jax.ShapeDtypeStruct((B, 1), jnp.float32),
            jax.ShapeDtypeStruct((B, 1), jnp.float32),
        ],
    )(p1, p2, plin, w1t, b1r, w2t, b2r, w3t, b3r)


def _tc_outer(s1, s2t):
    RB = 256

    def body(s1_ref, s2_ref, out_ref):
        out_ref[...] = jax.nn.sigmoid(s1_ref[...] + s2_ref[...])

    return pl.pallas_call(
        body,
        grid=(B // RB,),
        in_specs=[
            pl.BlockSpec((RB, 1), lambda i: (i, 0)),
            pl.BlockSpec((1, B), lambda i: (0, 0)),
        ],
        out_specs=pl.BlockSpec((RB, B), lambda i: (i, 0)),
        out_shape=jax.ShapeDtypeStruct((B, B), jnp.float32),
    )(s1, s2t)


def kernel(x, lin_w, lin_b, tables, w1, b1, w2, b2, w3, b3):
    x = x.astype(jnp.int32)
    offs = (jnp.arange(F, dtype=jnp.int32) * V)[None, :]
    xo = x + offs
    pi = jnp.asarray(_PI)
    pj = jnp.asarray(_PJ)
    idx1 = pj[None, :] * TOT + jnp.take(xo, pi, axis=1)
    idx2 = pi[None, :] * TOT + jnp.take(xo, pj, axis=1)
    pad = jnp.zeros((B, NPP - NP0), jnp.int32)
    idxp = jnp.concatenate([
        jnp.concatenate([idx1, pad], axis=1),
        jnp.concatenate([idx2, pad], axis=1),
    ], axis=0).reshape(-1)
    idxl = jnp.concatenate(
        [xo, jnp.full((B, FP - F), ZROW, jnp.int32)], axis=1).reshape(-1)
    tflat = tables.reshape(F * TOT, D)
    linpad = jnp.zeros((TOT + 8, D), jnp.float32).at[:TOT, 0].set(
        lin_w.reshape(TOT))

    pall, plin = _sc_gather(tflat, linpad, idxp, idxl)
    half = B * NPP
    p1 = lax.slice_in_dim(pall, 0, half, axis=0)
    p2 = lax.slice_in_dim(pall, half, 2 * half, axis=0)

    s1, s2 = _tc_mlp(p1, p2, plin, w1.T, b1[None, :], w2.T, b2[None, :],
                     w3.T, b3[None, :])

    s1 = s1 + lin_b[0]
    cpad = (w3 @ jax.nn.relu(w2 @ jax.nn.relu(b1) + b2) + b3)[0]
    s2 = s2 - (NPP - NP0) * cpad

    return _tc_outer(s1, s2.reshape(1, B))


# speedup vs baseline: 2.3306x; 2.3306x over previous
import functools

import jax
import jax.numpy as jnp
import numpy as np
from jax import lax
from jax.experimental import pallas as pl
from jax.experimental.pallas import tpu as pltpu
from jax.experimental.pallas import tpu_sc as plsc

F = 26
V = 1000
TOT = F * V
D = 16
B = 4096
NP0 = F * (F - 1) // 2
NPP = 328
RD = NPP * D // 128
FP = 32
LD = FP * D // 128
ZROW = TOT
_PI = np.array([i for i in range(F - 1) for j in range(i + 1, F)], dtype=np.int32)
_PJ = np.array([j for i in range(F - 1) for j in range(i + 1, F)], dtype=np.int32)

NC, NS = 2, 16
NW = NC * NS
PROWS = 2 * B * NPP
PRPW = PROWS // NW
PCHUNK = 2624
PNCHUNK = PRPW // PCHUNK
LROWS = B * FP
LRPW = LROWS // NW
LCHUNK = 1024
LNCHUNK = LRPW // LCHUNK


def _repack(src_ref, dst_ref, nrows):
    def step(r, _):
        for k in range(8):
            dst_ref[r, pl.ds(16 * k, 16)] = src_ref[8 * r + k, :]
        return 0

    lax.fori_loop(0, nrows // 8, step, 0)


def _sc_gather(tflat, linpad, idxp, idxl):
    @functools.partial(
        pl.kernel,
        out_type=(
            jax.ShapeDtypeStruct((PROWS * D // 128, 128), jnp.float32),
            jax.ShapeDtypeStruct((LROWS * D // 128, 128), jnp.float32),
        ),
        mesh=plsc.VectorSubcoreMesh(core_axis_name="c", subcore_axis_name="s",
                                    num_cores=NC, num_subcores=NS),
        compiler_params=pltpu.CompilerParams(use_tc_tiling_on_sc=False),
        scratch_types=[
            pltpu.VMEM((PCHUNK,), jnp.int32),
            pltpu.VMEM((PCHUNK, D), jnp.float32),
            pltpu.VMEM((PCHUNK * D // 128, 128), jnp.float32),
            pltpu.VMEM((LCHUNK,), jnp.int32),
            pltpu.VMEM((LCHUNK, D), jnp.float32),
            pltpu.SemaphoreType.DMA,
        ],
    )
    def body(tflat_hbm, linpad_hbm, idxp_hbm, idxl_hbm, pall_hbm, plin_hbm,
             idx_v, rows_v, wide_v, idxl_v, rowsl_v, sem):
        wid = lax.axis_index("s") * NC + lax.axis_index("c")
        base0 = wid * PRPW
        for c in range(PNCHUNK):
            base = base0 + c * PCHUNK
            pltpu.sync_copy(idxp_hbm.at[pl.ds(base, PCHUNK)], idx_v)
            pltpu.async_copy(tflat_hbm.at[idx_v], rows_v, sem).wait()
            _repack(rows_v, wide_v, PCHUNK)
            pltpu.sync_copy(
                wide_v,
                pall_hbm.at[pl.ds(base * D // 128, PCHUNK * D // 128)])
        lbase0 = wid * LRPW
        for c in range(LNCHUNK):
            base = lbase0 + c * LCHUNK
            pltpu.sync_copy(idxl_hbm.at[pl.ds(base, LCHUNK)], idxl_v)
            pltpu.async_copy(linpad_hbm.at[idxl_v], rowsl_v, sem).wait()
            _repack(rowsl_v, wide_v, LCHUNK)
            pltpu.sync_copy(
                wide_v.at[pl.ds(0, LCHUNK * D // 128)],
                plin_hbm.at[pl.ds(base * D // 128, LCHUNK * D // 128)])

    return body(tflat, linpad, idxp, idxl)


def _tc_mlp(p1d, p2d, plind, w1b, b1b, w2b, b2b, w3b, b3b):
    BB = 128
    NB = B // BB

    def body(p1_ref, p2_ref, plin_ref, w1_ref, b1_ref, w2_ref, b2_ref,
             w3_ref, b3_ref, s1_ref, s2_ref):
        ix = p1_ref[...] * p2_ref[...]
        ix3 = ix.reshape(BB, RD, 128)
        ii = lax.broadcasted_iota(jnp.int32, (BB, RD, 128), 1)
        jj = lax.broadcasted_iota(jnp.int32, (BB, RD, 128), 2)
        dummy = (ii == RD - 1) & (jj >= 128 - (NPP - NP0) * D)
        ixm = jnp.where(dummy, 0.0, ix3)
        ffm = jnp.sum(jnp.sum(ixm, axis=1), axis=1, keepdims=True)
        lin3 = plin_ref[...].reshape(BB, LD, 128)
        lin = jnp.sum(jnp.sum(lin3, axis=1), axis=1, keepdims=True)
        xmat = ixm.reshape(BB * RD, 128)
        h1 = jax.nn.relu(
            jnp.dot(xmat, w1_ref[...], preferred_element_type=jnp.float32)
            + b1_ref[...])
        h2 = jax.nn.relu(
            jnp.dot(h1, w2_ref[...], preferred_element_type=jnp.float32)
            + b2_ref[...])
        fi = (jnp.dot(h2, w3_ref[...], preferred_element_type=jnp.float32)
              + b3_ref[...])
        fi3 = fi.reshape(BB, RD, 8)
        fisum = jnp.sum(jnp.sum(fi3, axis=1), axis=1, keepdims=True)
        s1_ref[...] = lin + ffm
        s2_ref[...] = fisum

    return pl.pallas_call(
        body,
        grid=(NB,),
        in_specs=[
            pl.BlockSpec((BB * RD, 128), lambda i: (i, 0)),
            pl.BlockSpec((BB * RD, 128), lambda i: (i, 0)),
            pl.BlockSpec((BB * LD, 128), lambda i: (i, 0)),
            pl.BlockSpec((128, 512), lambda i: (0, 0)),
            pl.BlockSpec((1, 512), lambda i: (0, 0)),
            pl.BlockSpec((512, 256), lambda i: (0, 0)),
            pl.BlockSpec((1, 256), lambda i: (0, 0)),
            pl.BlockSpec((256, 8), lambda i: (0, 0)),
            pl.BlockSpec((1, 8), lambda i: (0, 0)),
        ],
        out_specs=[
            pl.BlockSpec((BB, 1), lambda i: (i, 0)),
            pl.BlockSpec((BB, 1), lambda i: (i, 0)),
        ],
        out_shape=[
            jax.ShapeDtypeStruct((B, 1), jnp.float32),
            jax.ShapeDtypeStruct((B, 1), jnp.float32),
        ],
    )(p1d, p2d, plind, w1b, b1b, w2b, b2b, w3b, b3b)


def _tc_outer(s1, s2t):
    RB = 256

    def body(s1_ref, s2_ref, out_ref):
        out_ref[...] = jax.nn.sigmoid(s1_ref[...] + s2_ref[...])

    return pl.pallas_call(
        body,
        grid=(B // RB,),
        in_specs=[
            pl.BlockSpec((RB, 1), lambda i: (i, 0)),
            pl.BlockSpec((1, B), lambda i: (0, 0)),
        ],
        out_specs=pl.BlockSpec((RB, B), lambda i: (i, 0)),
        out_shape=jax.ShapeDtypeStruct((B, B), jnp.float32),
    )(s1, s2t)


def kernel(x, lin_w, lin_b, tables, w1, b1, w2, b2, w3, b3):
    x = x.astype(jnp.int32)
    offs = (jnp.arange(F, dtype=jnp.int32) * V)[None, :]
    xo = x + offs
    pi = jnp.asarray(_PI)
    pj = jnp.asarray(_PJ)
    idx1 = pj[None, :] * TOT + jnp.take(xo, pi, axis=1)
    idx2 = pi[None, :] * TOT + jnp.take(xo, pj, axis=1)
    pad = jnp.zeros((B, NPP - NP0), jnp.int32)
    idxp = jnp.concatenate([
        jnp.concatenate([idx1, pad], axis=1),
        jnp.concatenate([idx2, pad], axis=1),
    ], axis=0).reshape(-1)
    idxl = jnp.concatenate(
        [xo, jnp.full((B, FP - F), ZROW, jnp.int32)], axis=1).reshape(-1)
    tflat = tables.reshape(F * TOT, D)
    linpad = jnp.zeros((TOT + 8, D), jnp.float32).at[:TOT, 0].set(
        lin_w.reshape(TOT))

    pall, plin = _sc_gather(tflat, linpad, idxp, idxl)
    half = B * RD
    p1d = lax.slice_in_dim(pall, 0, half, axis=0)
    p2d = lax.slice_in_dim(pall, half, 2 * half, axis=0)

    eye8 = jnp.eye(8, dtype=jnp.float32)
    w1b = jnp.kron(eye8, w1.T)
    w2b = jnp.kron(eye8, w2.T)
    w3b = jnp.kron(eye8, w3.T)
    b1b = jnp.tile(b1, 8)[None, :]
    b2b = jnp.tile(b2, 8)[None, :]
    b3b = jnp.tile(b3, 8)[None, :]

    s1, s2 = _tc_mlp(p1d, p2d, plin, w1b, b1b, w2b, b2b, w3b, b3b)

    s1 = s1 + lin_b[0]
    cpad = (w3 @ jax.nn.relu(w2 @ jax.nn.relu(b1) + b2) + b3)[0]
    s2 = s2 - (NPP - NP0) * cpad

    return _tc_outer(s1, s2.reshape(1, B))


# speedup vs baseline: 3.1395x; 1.3470x over previous
import functools

import jax
import jax.numpy as jnp
import numpy as np
from jax import lax
from jax.experimental import pallas as pl
from jax.experimental.pallas import tpu as pltpu
from jax.experimental.pallas import tpu_sc as plsc

F = 26
V = 1000
TOT = F * V
D = 16
B = 4096
NP0 = F * (F - 1) // 2
NPP = 328
RD = NPP * D // 128
FP = 32
LD = FP * D // 128
ZROW = TOT
_PI = np.array([i for i in range(F - 1) for j in range(i + 1, F)], dtype=np.int32)
_PJ = np.array([j for i in range(F - 1) for j in range(i + 1, F)], dtype=np.int32)

NC, NS = 2, 16
NW = NC * NS
HROWS = B * NPP
HRPW = HROWS // NW
CH = 1024
NCHUNK = HRPW // CH
CW = CH * D // 128
LRPW = B * FP // NW
LNCH = LRPW // CH


def _sc_gather(tflat, linpad, idxp, idxl):
    @functools.partial(
        pl.kernel,
        out_type=(
            jax.ShapeDtypeStruct((HROWS * D // 128, 128), jnp.float32),
            jax.ShapeDtypeStruct((B * FP * D // 128, 128), jnp.float32),
        ),
        mesh=plsc.VectorSubcoreMesh(core_axis_name="c", subcore_axis_name="s",
                                    num_cores=NC, num_subcores=NS),
        compiler_params=pltpu.CompilerParams(use_tc_tiling_on_sc=False),
        scratch_types=[
            pltpu.VMEM((CH,), jnp.int32),
            pltpu.VMEM((CH,), jnp.int32),
            pltpu.VMEM((CH,), jnp.int32),
            pltpu.VMEM((CH,), jnp.int32),
            pltpu.VMEM((CH, D), jnp.float32),
            pltpu.VMEM((CH, D), jnp.float32),
            pltpu.VMEM((CH, D), jnp.float32),
            pltpu.VMEM((CH, D), jnp.float32),
            pltpu.VMEM((CW, 128), jnp.float32),
            pltpu.VMEM((CW, 128), jnp.float32),
            pltpu.SemaphoreType.DMA,
            pltpu.SemaphoreType.DMA,
            pltpu.SemaphoreType.DMA,
            pltpu.SemaphoreType.DMA,
        ],
    )
    def body(tflat_hbm, linpad_hbm, idxp_hbm, idxl_hbm, ix_hbm, plin_hbm,
             idx1a, idx1b, idx2a, idx2b, r1a, r2a, r1b, r2b, wa, wb,
             sga, sgb, swa, swb):
        wid = lax.axis_index("s") * NC + lax.axis_index("c")
        base0 = wid * HRPW
        idx1_v = (idx1a, idx1b)
        idx2_v = (idx2a, idx2b)
        r1_v = (r1a, r1b)
        r2_v = (r2a, r2b)
        w_v = (wa, wb)
        sg = (sga, sgb)
        sw = (swa, swb)

        def fire(c):
            p = c % 2
            base = base0 + c * CH
            pltpu.sync_copy(idxp_hbm.at[pl.ds(base, CH)], idx1_v[p])
            g1 = pltpu.async_copy(tflat_hbm.at[idx1_v[p]], r1_v[p], sg[p])
            pltpu.sync_copy(idxp_hbm.at[pl.ds(HROWS + base, CH)], idx2_v[p])
            g2 = pltpu.async_copy(tflat_hbm.at[idx2_v[p]], r2_v[p], sg[p])
            return g1, g2

        def mulpack(r1, r2, w):
            def step(r, _):
                for k in range(8):
                    w[r, pl.ds(16 * k, 16)] = (
                        r1[8 * r + k, :] * r2[8 * r + k, :])
                return 0
            lax.fori_loop(0, CW, step, 0)

        pend = fire(0)
        wpend = [None, None]
        for c in range(NCHUNK):
            p = c % 2
            nxt = fire(c + 1) if c + 1 < NCHUNK else None
            pend[0].wait()
            pend[1].wait()
            pend = nxt
            if wpend[p] is not None:
                wpend[p].wait()
            mulpack(r1_v[p], r2_v[p], w_v[p])
            wpend[p] = pltpu.async_copy(
                w_v[p],
                ix_hbm.at[pl.ds((base0 + c * CH) * D // 128, CW)], sw[p])
        for p in range(2):
            if wpend[p] is not None:
                wpend[p].wait()

        lbase0 = wid * LRPW
        for c in range(LNCH):
            base = lbase0 + c * CH
            pltpu.sync_copy(idxl_hbm.at[pl.ds(base, CH)], idx1a)
            pltpu.async_copy(linpad_hbm.at[idx1a], r1a, sga).wait()

            def lstep(r, _):
                for k in range(8):
                    wa[r, pl.ds(16 * k, 16)] = r1a[8 * r + k, :]
                return 0
            lax.fori_loop(0, CW, lstep, 0)
            pltpu.sync_copy(wa, plin_hbm.at[pl.ds(base * D // 128, CW)])

    return body(tflat, linpad, idxp, idxl)


def _tc_mlp(ixd, plind, w1b, b1b, w2b, b2b, w3b, b3b):
    BB = 128
    NB = B // BB

    def body(ix_ref, plin_ref, w1_ref, b1_ref, w2_ref, b2_ref,
             w3_ref, b3_ref, s1_ref, s2_ref):
        ix3 = ix_ref[...].reshape(BB, RD, 128)
        ii = lax.broadcasted_iota(jnp.int32, (BB, RD, 128), 1)
        jj = lax.broadcasted_iota(jnp.int32, (BB, RD, 128), 2)
        dummy = (ii == RD - 1) & (jj >= 128 - (NPP - NP0) * D)
        ixm = jnp.where(dummy, 0.0, ix3)
        ffm = jnp.sum(jnp.sum(ixm, axis=1), axis=1, keepdims=True)
        lin3 = plin_ref[...].reshape(BB, LD, 128)
        lin = jnp.sum(jnp.sum(lin3, axis=1), axis=1, keepdims=True)
        xmat = ixm.reshape(BB * RD, 128).astype(jnp.bfloat16)
        h1 = jax.nn.relu(
            jnp.dot(xmat, w1_ref[...], preferred_element_type=jnp.float32)
            + b1_ref[...]).astype(jnp.bfloat16)
        h2 = jax.nn.relu(
            jnp.dot(h1, w2_ref[...], preferred_element_type=jnp.float32)
            + b2_ref[...]).astype(jnp.bfloat16)
        fi = (jnp.dot(h2, w3_ref[...], preferred_element_type=jnp.float32)
              + b3_ref[...])
        fi3 = fi.reshape(BB, RD, 8)
        fisum = jnp.sum(jnp.sum(fi3, axis=1), axis=1, keepdims=True)
        s1_ref[...] = lin + ffm
        s2_ref[...] = fisum

    return pl.pallas_call(
        body,
        grid=(NB,),
        in_specs=[
            pl.BlockSpec((BB * RD, 128), lambda i: (i, 0)),
            pl.BlockSpec((BB * LD, 128), lambda i: (i, 0)),
            pl.BlockSpec((128, 512), lambda i: (0, 0)),
            pl.BlockSpec((1, 512), lambda i: (0, 0)),
            pl.BlockSpec((512, 256), lambda i: (0, 0)),
            pl.BlockSpec((1, 256), lambda i: (0, 0)),
            pl.BlockSpec((256, 8), lambda i: (0, 0)),
            pl.BlockSpec((1, 8), lambda i: (0, 0)),
        ],
        out_specs=[
            pl.BlockSpec((BB, 1), lambda i: (i, 0)),
            pl.BlockSpec((BB, 1), lambda i: (i, 0)),
        ],
        out_shape=[
            jax.ShapeDtypeStruct((B, 1), jnp.float32),
            jax.ShapeDtypeStruct((B, 1), jnp.float32),
        ],
    )(ixd, plind, w1b, b1b, w2b, b2b, w3b, b3b)


def _tc_outer(s1, s2t):
    RB = 256

    def body(s1_ref, s2_ref, out_ref):
        out_ref[...] = jax.nn.sigmoid(s1_ref[...] + s2_ref[...])

    return pl.pallas_call(
        body,
        grid=(B // RB,),
        in_specs=[
            pl.BlockSpec((RB, 1), lambda i: (i, 0)),
            pl.BlockSpec((1, B), lambda i: (0, 0)),
        ],
        out_specs=pl.BlockSpec((RB, B), lambda i: (i, 0)),
        out_shape=jax.ShapeDtypeStruct((B, B), jnp.float32),
    )(s1, s2t)


def kernel(x, lin_w, lin_b, tables, w1, b1, w2, b2, w3, b3):
    x = x.astype(jnp.int32)
    offs = (jnp.arange(F, dtype=jnp.int32) * V)[None, :]
    xo = x + offs
    c1 = np.concatenate([_PI, np.zeros(NPP - NP0, np.int32)])
    a1 = np.concatenate([_PJ * TOT, np.zeros(NPP - NP0, np.int32)])
    c2 = np.concatenate([_PJ, np.zeros(NPP - NP0, np.int32)])
    a2 = np.concatenate([_PI * TOT, np.zeros(NPP - NP0, np.int32)])
    idx1 = jnp.take(xo, jnp.asarray(c1), axis=1) + jnp.asarray(a1)[None, :]
    idx2 = jnp.take(xo, jnp.asarray(c2), axis=1) + jnp.asarray(a2)[None, :]
    idxp = jnp.concatenate([idx1, idx2], axis=0).reshape(-1)
    idxl = jnp.concatenate(
        [xo, jnp.full((B, FP - F), ZROW, jnp.int32)], axis=1).reshape(-1)
    tflat = tables.reshape(F * TOT, D)
    linpad = jnp.zeros((TOT + 8, D), jnp.float32).at[:TOT, 0].set(
        lin_w.reshape(TOT))

    ixd, plin = _sc_gather(tflat, linpad, idxp, idxl)

    eye8 = jnp.eye(8, dtype=jnp.bfloat16)
    w1b = jnp.kron(eye8, w1.T.astype(jnp.bfloat16))
    w2b = jnp.kron(eye8, w2.T.astype(jnp.bfloat16))
    w3b = jnp.kron(eye8, w3.T.astype(jnp.bfloat16))
    b1b = jnp.tile(b1, 8)[None, :]
    b2b = jnp.tile(b2, 8)[None, :]
    b3b = jnp.tile(b3, 8)[None, :]

    s1, s2 = _tc_mlp(ixd, plin, w1b, b1b, w2b, b2b, w3b, b3b)

    s1 = s1 + lin_b[0]
    cpad = (w3 @ jax.nn.relu(w2 @ jax.nn.relu(b1) + b2) + b3)[0]
    s2 = s2 - (NPP - NP0) * cpad

    return _tc_outer(s1, s2.reshape(1, B))


# speedup vs baseline: 3.4793x; 1.1082x over previous
import functools

import jax
import jax.numpy as jnp
import numpy as np
from jax import lax
from jax.experimental import pallas as pl
from jax.experimental.pallas import tpu as pltpu
from jax.experimental.pallas import tpu_sc as plsc

F = 26
V = 1000
TOT = F * V
D = 16
B = 4096
NP0 = F * (F - 1) // 2
NPP = 328
RD = NPP * D // 128
FP = 32
LD = FP * D // 128
ZROW = TOT
_PI = np.array([i for i in range(F - 1) for j in range(i + 1, F)], dtype=np.int32)
_PJ = np.array([j for i in range(F - 1) for j in range(i + 1, F)], dtype=np.int32)

NC, NS = 2, 16
NW = NC * NS
HROWS = B * NPP
HRPW = HROWS // NW
CH = 1024
NCHUNK = HRPW // CH
CW = CH * D // 128
LRPW = B * FP // NW
LNCH = LRPW // CH


def _sc_gather(tflat, linpad, idxp, idxl):
    @functools.partial(
        pl.kernel,
        out_type=(
            jax.ShapeDtypeStruct((HROWS * D // 128, 128), jnp.float32),
            jax.ShapeDtypeStruct((B * FP * D // 128, 128), jnp.float32),
        ),
        mesh=plsc.VectorSubcoreMesh(core_axis_name="c", subcore_axis_name="s",
                                    num_cores=NC, num_subcores=NS),
        compiler_params=pltpu.CompilerParams(use_tc_tiling_on_sc=False),
        scratch_types=[
            pltpu.VMEM((CH,), jnp.int32),
            pltpu.VMEM((CH,), jnp.int32),
            pltpu.VMEM((CH,), jnp.int32),
            pltpu.VMEM((CH,), jnp.int32),
            pltpu.VMEM((CH, D), jnp.float32),
            pltpu.VMEM((CH, D), jnp.float32),
            pltpu.VMEM((CH, D), jnp.float32),
            pltpu.VMEM((CH, D), jnp.float32),
            pltpu.VMEM((CW, 128), jnp.float32),
            pltpu.VMEM((CW, 128), jnp.float32),
            pltpu.SemaphoreType.DMA,
            pltpu.SemaphoreType.DMA,
            pltpu.SemaphoreType.DMA,
            pltpu.SemaphoreType.DMA,
        ],
    )
    def body(tflat_hbm, linpad_hbm, idxp_hbm, idxl_hbm, ix_hbm, plin_hbm,
             idx1a, idx1b, idx2a, idx2b, r1a, r2a, r1b, r2b, wa, wb,
             sga, sgb, swa, swb):
        wid = lax.axis_index("s") * NC + lax.axis_index("c")
        base0 = wid * HRPW
        idx1_v = (idx1a, idx1b)
        idx2_v = (idx2a, idx2b)
        r1_v = (r1a, r1b)
        r2_v = (r2a, r2b)
        w_v = (wa, wb)
        sg = (sga, sgb)
        sw = (swa, swb)

        def fire(c):
            p = c % 2
            base = base0 + c * CH
            pltpu.sync_copy(idxp_hbm.at[pl.ds(base, CH)], idx1_v[p])
            g1 = pltpu.async_copy(tflat_hbm.at[idx1_v[p]], r1_v[p], sg[p])
            pltpu.sync_copy(idxp_hbm.at[pl.ds(HROWS + base, CH)], idx2_v[p])
            g2 = pltpu.async_copy(tflat_hbm.at[idx2_v[p]], r2_v[p], sg[p])
            return g1, g2

        def mulpack(r1, r2, w):
            def step(r, _):
                for k in range(8):
                    w[r, pl.ds(16 * k, 16)] = (
                        r1[8 * r + k, :] * r2[8 * r + k, :])
                return 0
            lax.fori_loop(0, CW, step, 0)

        pend = fire(0)
        wpend = [None, None]
        for c in range(NCHUNK):
            p = c % 2
            nxt = fire(c + 1) if c + 1 < NCHUNK else None
            pend[0].wait()
            pend[1].wait()
            pend = nxt
            if wpend[p] is not None:
                wpend[p].wait()
            mulpack(r1_v[p], r2_v[p], w_v[p])
            wpend[p] = pltpu.async_copy(
                w_v[p],
                ix_hbm.at[pl.ds((base0 + c * CH) * D // 128, CW)], sw[p])
        for p in range(2):
            if wpend[p] is not None:
                wpend[p].wait()

        lbase0 = wid * LRPW
        for c in range(LNCH):
            base = lbase0 + c * CH
            pltpu.sync_copy(idxl_hbm.at[pl.ds(base, CH)], idx1a)
            pltpu.async_copy(linpad_hbm.at[idx1a], r1a, sga).wait()

            def lstep(r, _):
                for k in range(8):
                    wa[r, pl.ds(16 * k, 16)] = r1a[8 * r + k, :]
                return 0
            lax.fori_loop(0, CW, lstep, 0)
            pltpu.sync_copy(wa, plin_hbm.at[pl.ds(base * D // 128, CW)])

    return body(tflat, linpad, idxp, idxl)


def _tc_mlp(ixd, plind, maskm, gsel, gsel2, ones1, ones8,
            w1b, b1b, w2b, b2b, w3b, b3b):
    BB = 128
    NB = B // BB

    def body(ix_ref, plin_ref, mask_ref, g_ref, g2_ref, o1_ref, o8_ref,
             w1_ref, b1_ref, w2_ref, b2_ref, w3_ref, b3_ref,
             s1_ref, s2_ref):
        ixm = ix_ref[...] * mask_ref[...]
        rs = jnp.dot(ixm, o1_ref[...],
                     preferred_element_type=jnp.float32)
        ffm = jnp.dot(g_ref[...], rs,
                      preferred_element_type=jnp.float32)
        rs2 = jnp.dot(plin_ref[...], o1_ref[...],
                      preferred_element_type=jnp.float32)
        lin = jnp.dot(g2_ref[...], rs2,
                      preferred_element_type=jnp.float32)
        xmat = ixm.astype(jnp.bfloat16)
        h1 = jax.nn.relu(
            jnp.dot(xmat, w1_ref[...], preferred_element_type=jnp.float32)
            + b1_ref[...]).astype(jnp.bfloat16)
        h2 = jax.nn.relu(
            jnp.dot(h1, w2_ref[...], preferred_element_type=jnp.float32)
            + b2_ref[...]).astype(jnp.bfloat16)
        fi = (jnp.dot(h2, w3_ref[...], preferred_element_type=jnp.float32)
              + b3_ref[...])
        fr = jnp.dot(fi, o8_ref[...],
                     preferred_element_type=jnp.float32)
        fisum = jnp.dot(g_ref[...], fr,
                        preferred_element_type=jnp.float32)
        s1_ref[...] = lin + ffm
        s2_ref[...] = fisum

    return pl.pallas_call(
        body,
        grid=(NB,),
        in_specs=[
            pl.BlockSpec((BB * RD, 128), lambda i: (i, 0)),
            pl.BlockSpec((BB * LD, 128), lambda i: (i, 0)),
            pl.BlockSpec((BB * RD, 128), lambda i: (0, 0)),
            pl.BlockSpec((BB, BB * RD), lambda i: (0, 0)),
            pl.BlockSpec((BB, BB * LD), lambda i: (0, 0)),
            pl.BlockSpec((128, 1), lambda i: (0, 0)),
            pl.BlockSpec((8, 1), lambda i: (0, 0)),
            pl.BlockSpec((128, 512), lambda i: (0, 0)),
            pl.BlockSpec((1, 512), lambda i: (0, 0)),
            pl.BlockSpec((512, 256), lambda i: (0, 0)),
            pl.BlockSpec((1, 256), lambda i: (0, 0)),
            pl.BlockSpec((256, 8), lambda i: (0, 0)),
            pl.BlockSpec((1, 8), lambda i: (0, 0)),
        ],
        out_specs=[
            pl.BlockSpec((BB, 1), lambda i: (i, 0)),
            pl.BlockSpec((BB, 1), lambda i: (i, 0)),
        ],
        out_shape=[
            jax.ShapeDtypeStruct((B, 1), jnp.float32),
            jax.ShapeDtypeStruct((B, 1), jnp.float32),
        ],
    )(ixd, plind, maskm, gsel, gsel2, ones1, ones8,
      w1b, b1b, w2b, b2b, w3b, b3b)


def _tc_outer(s1, s2t):
    RB = 256

    def body(s1_ref, s2_ref, out_ref):
        out_ref[...] = jax.nn.sigmoid(s1_ref[...] + s2_ref[...])

    return pl.pallas_call(
        body,
        grid=(B // RB,),
        in_specs=[
            pl.BlockSpec((RB, 1), lambda i: (i, 0)),
            pl.BlockSpec((1, B), lambda i: (0, 0)),
        ],
        out_specs=pl.BlockSpec((RB, B), lambda i: (i, 0)),
        out_shape=jax.ShapeDtypeStruct((B, B), jnp.float32),
    )(s1, s2t)


def kernel(x, lin_w, lin_b, tables, w1, b1, w2, b2, w3, b3):
    x = x.astype(jnp.int32)
    offs = (jnp.arange(F, dtype=jnp.int32) * V)[None, :]
    xo = x + offs
    c1 = np.concatenate([_PI, np.zeros(NPP - NP0, np.int32)])
    a1 = np.concatenate([_PJ * TOT, np.zeros(NPP - NP0, np.int32)])
    c2 = np.concatenate([_PJ, np.zeros(NPP - NP0, np.int32)])
    a2 = np.concatenate([_PI * TOT, np.zeros(NPP - NP0, np.int32)])
    idx1 = jnp.take(xo, jnp.asarray(c1), axis=1) + jnp.asarray(a1)[None, :]
    idx2 = jnp.take(xo, jnp.asarray(c2), axis=1) + jnp.asarray(a2)[None, :]
    idxp = jnp.concatenate([idx1, idx2], axis=0).reshape(-1)
    idxl = jnp.concatenate(
        [xo, jnp.full((B, FP - F), ZROW, jnp.int32)], axis=1).reshape(-1)
    linpad = jnp.zeros((TOT + 8, D), jnp.float32).at[:TOT, 0].set(
        lin_w.reshape(TOT))

    ixd, plin = _sc_gather(tables.reshape(F * TOT, D), linpad, idxp, idxl)

    BB = 128
    mrow = np.ones((RD, 128), np.float32)
    mrow[RD - 1, 128 - (NPP - NP0) * D:] = 0.0
    maskm = jnp.asarray(np.tile(mrow, (BB, 1)))
    gs = np.zeros((BB, BB * RD), np.float32)
    gs[np.arange(BB * RD) // RD, np.arange(BB * RD)] = 1.0
    gsel = jnp.asarray(gs)
    gs2 = np.zeros((BB, BB * LD), np.float32)
    gs2[np.arange(BB * LD) // LD, np.arange(BB * LD)] = 1.0
    gsel2 = jnp.asarray(gs2)
    ones1 = jnp.ones((128, 1), jnp.float32)
    ones8 = jnp.ones((8, 1), jnp.float32)

    eye8 = jnp.eye(8, dtype=jnp.bfloat16)
    w1b = jnp.kron(eye8, w1.T.astype(jnp.bfloat16))
    w2b = jnp.kron(eye8, w2.T.astype(jnp.bfloat16))
    w3b = jnp.kron(eye8, w3.T.astype(jnp.bfloat16))
    b1b = jnp.tile(b1, 8)[None, :]
    b2b = jnp.tile(b2, 8)[None, :]
    b3b = jnp.tile(b3, 8)[None, :]

    s1, s2 = _tc_mlp(ixd, plin, maskm, gsel, gsel2, ones1, ones8,
                     w1b, b1b, w2b, b2b, w3b, b3b)

    s1 = s1 + lin_b[0]
    cpad = (w3 @ jax.nn.relu(w2 @ jax.nn.relu(b1) + b2) + b3)[0]
    s2 = s2 - (NPP - NP0) * cpad

    return _tc_outer(s1, s2.reshape(1, B))
